# baseline (speedup 1.0000x reference)
import functools

import jax
import jax.numpy as jnp
from jax import lax
from jax.experimental import pallas as pl
from jax.experimental.pallas import tpu as pltpu
from jax.experimental.pallas import tpu_sc as plsc

B = 16384
D = 64
NC = 2
NS = 16
NW = NC * NS
BPW = B // NW
L = 16


def _sc_scores(user_hbm, pos_hbm, neg_hbm, eu_hbm, ei_hbm, out_hbm,
               idx_u, idx_p, idx_n, u_v, p_v, n_v, s_v, sem):
    wid = lax.axis_index("s") * NC + lax.axis_index("c")
    base = wid * BPW

    pltpu.sync_copy(user_hbm.at[pl.ds(base, BPW)], idx_u)
    pltpu.sync_copy(pos_hbm.at[pl.ds(base, BPW)], idx_p)
    pltpu.sync_copy(neg_hbm.at[pl.ds(base, BPW)], idx_n)

    for f in range(D):
        pltpu.async_copy(eu_hbm.at[f].at[idx_u], u_v.at[f], sem)
        pltpu.async_copy(ei_hbm.at[f].at[idx_p], p_v.at[f], sem)
        pltpu.async_copy(ei_hbm.at[f].at[idx_n], n_v.at[f], sem)

    for f in range(D):
        for buf in (u_v, p_v, n_v):
            pltpu.make_async_copy(eu_hbm.at[0].at[idx_u], buf.at[f], sem).wait()

    def body_c(g, carry):
        sl = pl.ds(g * L, L)
        tot = u_v[0, sl] * (p_v[0, sl] - n_v[0, sl])
        for f in range(1, D):
            tot = tot + u_v[f, sl] * (p_v[f, sl] - n_v[f, sl])
        s_v[sl] = tot
        return carry

    lax.fori_loop(0, BPW // L, body_c, 0)

    pltpu.sync_copy(s_v, out_hbm.at[pl.ds(base, BPW)])


_sc_scores_call = functools.partial(
    pl.kernel,
    out_type=jax.ShapeDtypeStruct((B,), jnp.float32),
    mesh=plsc.VectorSubcoreMesh(core_axis_name="c", subcore_axis_name="s",
                                num_cores=NC, num_subcores=NS),
    scratch_types=[
        pltpu.VMEM((BPW,), jnp.int32),
        pltpu.VMEM((BPW,), jnp.int32),
        pltpu.VMEM((BPW,), jnp.int32),
        pltpu.VMEM((D, BPW), jnp.float32),
        pltpu.VMEM((D, BPW), jnp.float32),
        pltpu.VMEM((D, BPW), jnp.float32),
        pltpu.VMEM((BPW,), jnp.float32),
        pltpu.SemaphoreType.DMA,
    ],
    compiler_params=pltpu.CompilerParams(needs_layout_passes=False,
                                         use_tc_tiling_on_sc=False),
    name="bpr_sc_scores",
)(_sc_scores)


def _tc_loss_body(s_ref, o_ref):
    s = s_ref[...]
    x = -s
    m = jnp.maximum(x, 0.0)
    sp = m + jnp.log(1.0 + jnp.exp(-jnp.abs(x)))
    o_ref[0, 0] = jnp.sum(sp) * (1.0 / B)


_tc_loss_call = pl.pallas_call(
    _tc_loss_body,
    out_shape=jax.ShapeDtypeStruct((1, 1), jnp.float32),
    in_specs=[pl.BlockSpec(memory_space=pltpu.VMEM)],
    out_specs=pl.BlockSpec(memory_space=pltpu.SMEM),
)


@jax.jit
def kernel(user, pos, neg, labels, embedding_user, embedding_item):
    del labels
    s = _sc_scores_call(user.astype(jnp.int32), pos.astype(jnp.int32),
                        neg.astype(jnp.int32),
                        embedding_user.T, embedding_item.T)
    loss = _tc_loss_call(s.reshape(B // 128, 128))
    return loss[0, 0]


# speedup vs baseline: 12.7162x; 12.7162x over previous
import functools

import jax
import jax.numpy as jnp
from jax import lax
from jax.experimental import pallas as pl
from jax.experimental.pallas import tpu as pltpu
from jax.experimental.pallas import tpu_sc as plsc

B = 16384
D = 64
NC = 2
NS = 16
NW = NC * NS
L = 16
NV = 1000000
BW = 128
NBLK = -(-NV // BW)
BPT = 248
SEGN = 8192
SEGW = 272
FLUSH = 128
SENT = 0x7FFFFFF0


def _harvest_pass(tab_hbm, n_sorted, sv, ev, segv, blk, stg, eid, out_hbm,
                  sems):
    lanes = lax.iota(jnp.int32, L)
    wid = lax.axis_index("s") * NC + lax.axis_index("c")
    b0 = wid * BPT
    nrows = out_hbm.shape[0] - FLUSH

    def fire(b, par):
        @pl.when(jnp.logical_and(b < BPT, b0 + b < NBLK))
        def _():
            off = pl.multiple_of((b0 + b) * BW, BW)
            pltpu.async_copy(tab_hbm.at[:, pl.ds(off, BW)], blk.at[par],
                             sems[par])

    def drain(b, par):
        @pl.when(jnp.logical_and(b < BPT, b0 + b < NBLK))
        def _():
            pltpu.make_async_copy(tab_hbm.at[:, pl.ds(0, BW)], blk.at[par],
                                  sems[par]).wait()

    def scatter():
        pltpu.async_copy(stg, out_hbm.at[eid.at[0]], sems[2]).wait()

    def sanitize(cnt):
        def body_s(i, c):
            pos = i * L + lanes
            cur = eid[0, pl.ds(i * L, L)]
            eid[0, pl.ds(i * L, L)] = jnp.where(pos >= cnt,
                                                nrows + pos, cur)
            return c
        lax.fori_loop(0, FLUSH // L, body_s, 0)

    def extract(par, p, cnt):
        v16 = sv[pl.ds(p, L)]
        e16 = ev[pl.ds(p, L)]
        c = v16[0] & jnp.int32(BW - 1)
        cv = jnp.full((L,), c, jnp.int32)
        for k in range(D // L):
            g = plsc.load_gather(blk.at[par], [k * L + lanes, cv])
            stg[cnt, pl.ds(k * L, L)] = g
        eid[0, pl.ds(cnt, L)] = e16
        cnt = cnt + 1

        @pl.when(cnt == FLUSH)
        def _():
            scatter()

        return jnp.where(cnt == FLUSH, 0, cnt)

    def one_block(gg, b, par, cnt):
        drain(b, par)
        fire(b + 2, par)
        sv16 = segv[pl.ds(2 * gg, L)]
        s0 = sv16[par]
        s1 = sv16[par + 1]

        def body_e(p, c):
            return extract(par, p, c)

        return lax.fori_loop(s0, s1, body_e, cnt)

    fire(jnp.int32(0), 0)
    fire(jnp.int32(1), 1)

    def body_pair(gg, cnt):
        cnt = one_block(gg, 2 * gg, 0, cnt)
        cnt = one_block(gg, 2 * gg + 1, 1, cnt)
        return cnt

    cnt = lax.fori_loop(0, BPT // 2, body_pair, jnp.int32(0))
    sanitize(cnt)
    scatter()


def _sc_harvest(su, eu, segu, si, ei, segi, eu_hbm, ei_hbm,
                urows_hbm, irows_hbm,
                sv, ev, sgv, blk, stg, eid, sem0, sem1, sem2):
    wid = lax.axis_index("s") * NC + lax.axis_index("c")
    b0 = wid * BPT
    sems = (sem0, sem1, sem2)

    pltpu.sync_copy(su, sv.at[pl.ds(0, B + L)])
    pltpu.sync_copy(eu, ev.at[pl.ds(0, B + L)])
    pltpu.sync_copy(segu.at[pl.ds(b0, SEGW)], sgv)
    _harvest_pass(eu_hbm, B, sv, ev, sgv, blk, stg, eid, urows_hbm, sems)

    pltpu.sync_copy(si, sv)
    pltpu.sync_copy(ei, ev)
    pltpu.sync_copy(segi.at[pl.ds(b0, SEGW)], sgv)
    _harvest_pass(ei_hbm, 2 * B, sv, ev, sgv, blk, stg, eid, irows_hbm, sems)


_sc_harvest_call = functools.partial(
    pl.kernel,
    out_type=(jax.ShapeDtypeStruct((B + FLUSH, 2 * D), jnp.float32),
              jax.ShapeDtypeStruct((2 * B + FLUSH, 2 * D), jnp.float32)),
    mesh=plsc.VectorSubcoreMesh(core_axis_name="c", subcore_axis_name="s",
                                num_cores=NC, num_subcores=NS),
    scratch_types=[
        pltpu.VMEM((2 * B + L,), jnp.int32),
        pltpu.VMEM((2 * B + L,), jnp.int32),
        pltpu.VMEM((SEGW,), jnp.int32),
        pltpu.VMEM((2, D, BW), jnp.float32),
        pltpu.VMEM((FLUSH, 2 * D), jnp.float32),
        pltpu.VMEM((2, FLUSH), jnp.int32),
        pltpu.SemaphoreType.DMA,
        pltpu.SemaphoreType.DMA,
        pltpu.SemaphoreType.DMA,
    ],
    compiler_params=pltpu.CompilerParams(needs_layout_passes=False),
    name="bpr_sc_harvest",
)(_sc_harvest)


_TCB = 1024


def _tc_loss_body(u_ref, p_ref, n_ref, o_ref):
    i = pl.program_id(0)
    u = u_ref[:, :D]
    d = p_ref[:, :D] - n_ref[:, :D]
    s = jnp.sum(u * d, axis=-1)
    x = -s
    m = jnp.maximum(x, 0.0)
    sp = m + jnp.log(1.0 + jnp.exp(-jnp.abs(x)))
    part = jnp.sum(sp) * (1.0 / B)

    @pl.when(i == 0)
    def _():
        o_ref[0, 0] = 0.0

    o_ref[0, 0] += part


_tc_loss_call = pl.pallas_call(
    _tc_loss_body,
    out_shape=jax.ShapeDtypeStruct((1, 1), jnp.float32),
    grid=(B // _TCB,),
    in_specs=[
        pl.BlockSpec((_TCB, 2 * D), lambda i: (i, 0)),
        pl.BlockSpec((_TCB, 2 * D), lambda i: (i, 0)),
        pl.BlockSpec((_TCB, 2 * D), lambda i: (B // _TCB + i, 0)),
    ],
    out_specs=pl.BlockSpec(memory_space=pltpu.SMEM),
)


@jax.jit
def kernel(user, pos, neg, labels, embedding_user, embedding_item):
    del labels
    user = user.astype(jnp.int32)
    pos = pos.astype(jnp.int32)
    neg = neg.astype(jnp.int32)

    ou = jnp.argsort(user).astype(jnp.int32)
    su = user[ou]
    itv = jnp.concatenate([pos, neg])
    oi = jnp.argsort(itv).astype(jnp.int32)
    si = itv[oi]

    edges = (jnp.arange(SEGN, dtype=jnp.int32) * BW).clip(0, NV)
    segu = jnp.searchsorted(su, edges, side="left").astype(jnp.int32)
    segi = jnp.searchsorted(si, edges, side="left").astype(jnp.int32)

    pad_s = jnp.full((L,), jnp.int32(SENT), jnp.int32)
    su = jnp.concatenate([su, pad_s, jnp.full((B,), jnp.int32(SENT), jnp.int32)])
    ou = jnp.concatenate([ou, pad_s, jnp.full((B,), jnp.int32(SENT), jnp.int32)])
    si = jnp.concatenate([si, pad_s])
    oi = jnp.concatenate([oi, pad_s])

    u_rows, i_rows = _sc_harvest_call(su[:B + L], ou[:B + L], segu,
                                      si, oi, segi,
                                      embedding_user.T, embedding_item.T)
    loss = _tc_loss_call(u_rows, i_rows, i_rows)
    return loss[0, 0]


# speedup vs baseline: 15.0852x; 1.1863x over previous
import functools

import jax
import jax.numpy as jnp
from jax import lax
from jax.experimental import pallas as pl
from jax.experimental.pallas import tpu as pltpu
from jax.experimental.pallas import tpu_sc as plsc

B = 16384
D = 64
NC = 2
NS = 16
NW = NC * NS
L = 16
NV = 1000000
BW = 128
NBLK = -(-NV // BW)
BPT = 248
SEGN = 8192
SEGW = 272
FLUSH = 128
SENT = 0x7FFFFFF0


def _harvest_pass(tab_hbm, n_sorted, sv, ev, segv, blk, stg, eid, out_hbm,
                  sems):
    lanes = lax.iota(jnp.int32, L)
    wid = lax.axis_index("s") * NC + lax.axis_index("c")
    b0 = wid * BPT
    nrows = out_hbm.shape[0] - FLUSH

    def fire(b, par):
        @pl.when(jnp.logical_and(b < BPT, b0 + b < NBLK))
        def _():
            off = pl.multiple_of((b0 + b) * BW, BW)
            pltpu.async_copy(tab_hbm.at[:, pl.ds(off, BW)], blk.at[par],
                             sems[par])

    def drain(b, par):
        @pl.when(jnp.logical_and(b < BPT, b0 + b < NBLK))
        def _():
            pltpu.make_async_copy(tab_hbm.at[:, pl.ds(0, BW)], blk.at[par],
                                  sems[par]).wait()

    def scatter():
        pltpu.async_copy(stg, out_hbm.at[eid.at[0]], sems[2]).wait()

    def sanitize(cnt):
        def body_s(i, c):
            pos = i * L + lanes
            cur = eid[0, pl.ds(i * L, L)]
            eid[0, pl.ds(i * L, L)] = jnp.where(pos >= cnt,
                                                nrows + pos, cur)
            return c
        lax.fori_loop(0, FLUSH // L, body_s, 0)

    def extract(par, p, cnt):
        v16 = sv[pl.ds(p, L)]
        e16 = ev[pl.ds(p, L)]
        c = v16[0] & jnp.int32(BW - 1)
        cv = jnp.full((L,), c, jnp.int32)
        for k in range(D // L):
            g = plsc.load_gather(blk.at[par], [k * L + lanes, cv])
            stg[cnt, pl.ds(k * L, L)] = g
        eid[0, pl.ds(cnt, L)] = e16
        cnt = cnt + 1

        @pl.when(cnt == FLUSH)
        def _():
            scatter()

        return jnp.where(cnt == FLUSH, 0, cnt)

    def one_block(gg, b, par, cnt):
        drain(b, par)
        fire(b + 2, par)
        sv16 = segv[pl.ds(2 * gg, L)]
        s0 = sv16[par]
        s1 = sv16[par + 1]

        def body_e(p, c):
            return extract(par, p, c)

        return lax.fori_loop(s0, s1, body_e, cnt)

    fire(jnp.int32(0), 0)
    fire(jnp.int32(1), 1)

    def body_pair(gg, cnt):
        cnt = one_block(gg, 2 * gg, 0, cnt)
        cnt = one_block(gg, 2 * gg + 1, 1, cnt)
        return cnt

    cnt = lax.fori_loop(0, BPT // 2, body_pair, jnp.int32(0))
    sanitize(cnt)
    scatter()


def _sc_harvest(su, eu, segu, si, ei, segi, eu_hbm, ei_hbm,
                urows_hbm, irows_hbm,
                sv, ev, sgv, blk, stg, eid, sem0, sem1, sem2):
    wid = lax.axis_index("s") * NC + lax.axis_index("c")
    b0 = wid * BPT
    sems = (sem0, sem1, sem2)

    pltpu.sync_copy(su, sv.at[pl.ds(0, B + L)])
    pltpu.sync_copy(eu, ev.at[pl.ds(0, B + L)])
    pltpu.sync_copy(segu.at[pl.ds(b0, SEGW)], sgv)
    _harvest_pass(eu_hbm, B, sv, ev, sgv, blk, stg, eid, urows_hbm, sems)

    pltpu.sync_copy(si, sv)
    pltpu.sync_copy(ei, ev)
    pltpu.sync_copy(segi.at[pl.ds(b0, SEGW)], sgv)
    _harvest_pass(ei_hbm, 2 * B, sv, ev, sgv, blk, stg, eid, irows_hbm, sems)


_sc_harvest_call = functools.partial(
    pl.kernel,
    out_type=(jax.ShapeDtypeStruct((B + FLUSH, 2 * D), jnp.float32),
              jax.ShapeDtypeStruct((2 * B + FLUSH, 2 * D), jnp.float32)),
    mesh=plsc.VectorSubcoreMesh(core_axis_name="c", subcore_axis_name="s",
                                num_cores=NC, num_subcores=NS),
    scratch_types=[
        pltpu.VMEM((2 * B + L,), jnp.int32),
        pltpu.VMEM((2 * B + L,), jnp.int32),
        pltpu.VMEM((SEGW,), jnp.int32),
        pltpu.VMEM((2, D, BW), jnp.float32),
        pltpu.VMEM((FLUSH, 2 * D), jnp.float32),
        pltpu.VMEM((2, FLUSH), jnp.int32),
        pltpu.SemaphoreType.DMA,
        pltpu.SemaphoreType.DMA,
        pltpu.SemaphoreType.DMA,
    ],
    compiler_params=pltpu.CompilerParams(needs_layout_passes=False),
    name="bpr_sc_harvest",
)(_sc_harvest)


_TCB = 1024


def _tc_loss_body(u_ref, p_ref, n_ref, o_ref):
    i = pl.program_id(0)
    u = u_ref[:, :D]
    d = p_ref[:, :D] - n_ref[:, :D]
    s = jnp.sum(u * d, axis=-1)
    x = -s
    m = jnp.maximum(x, 0.0)
    sp = m + jnp.log(1.0 + jnp.exp(-jnp.abs(x)))
    part = jnp.sum(sp) * (1.0 / B)

    @pl.when(i == 0)
    def _():
        o_ref[0, 0] = 0.0

    o_ref[0, 0] += part


_tc_loss_call = pl.pallas_call(
    _tc_loss_body,
    out_shape=jax.ShapeDtypeStruct((1, 1), jnp.float32),
    grid=(B // _TCB,),
    in_specs=[
        pl.BlockSpec((_TCB, 2 * D), lambda i: (i, 0)),
        pl.BlockSpec((_TCB, 2 * D), lambda i: (i, 0)),
        pl.BlockSpec((_TCB, 2 * D), lambda i: (B // _TCB + i, 0)),
    ],
    out_specs=pl.BlockSpec(memory_space=pltpu.SMEM),
)


@jax.jit
def kernel(user, pos, neg, labels, embedding_user, embedding_item):
    del labels
    user = user.astype(jnp.int32)
    pos = pos.astype(jnp.int32)
    neg = neg.astype(jnp.int32)

    su, ou = lax.sort_key_val(user, jnp.arange(B, dtype=jnp.int32))
    itv = jnp.concatenate([pos, neg])
    si, oi = lax.sort_key_val(itv, jnp.arange(2 * B, dtype=jnp.int32))

    edges = (jnp.arange(SEGN, dtype=jnp.int32) * BW).clip(0, NV)
    segu = jnp.searchsorted(su, edges, side="left",
                            method="sort").astype(jnp.int32)
    segi = jnp.searchsorted(si, edges, side="left",
                            method="sort").astype(jnp.int32)

    pad_s = jnp.full((L,), jnp.int32(SENT), jnp.int32)
    su = jnp.concatenate([su, pad_s, jnp.full((B,), jnp.int32(SENT), jnp.int32)])
    ou = jnp.concatenate([ou, pad_s, jnp.full((B,), jnp.int32(SENT), jnp.int32)])
    si = jnp.concatenate([si, pad_s])
    oi = jnp.concatenate([oi, pad_s])

    u_rows, i_rows = _sc_harvest_call(su[:B + L], ou[:B + L], segu,
                                      si, oi, segi,
                                      embedding_user.T, embedding_item.T)
    loss = _tc_loss_call(u_rows, i_rows, i_rows)
    return loss[0, 0]


# speedup vs baseline: 28.6719x; 1.9007x over previous
import functools

import jax
import jax.numpy as jnp
from jax import lax
from jax.experimental import pallas as pl
from jax.experimental.pallas import tpu as pltpu
from jax.experimental.pallas import tpu_sc as plsc

B = 16384
D = 64
NC = 2
NS = 16
NW = NC * NS
L = 16
NV = 1000000
BW = 128
NBLK = -(-NV // BW)
BPT = 248
SEGN = 8192
SEGW = 272
FLUSH = 128
SENT = 0x7FFFFFF0


def _harvest_pass(tab_hbm, n_sorted, sv, ev, segv, blk, stg, eid, out_hbm,
                  sems, n_iters):
    lanes = lax.iota(jnp.int32, L)
    wid = lax.axis_index("s") * NC + lax.axis_index("c")
    b0 = wid * BPT
    nrows = out_hbm.shape[0] - FLUSH

    def fire(b, par):
        @pl.when(jnp.logical_and(b < BPT, b0 + b < NBLK))
        def _():
            off = pl.multiple_of((b0 + b) * BW, BW)
            pltpu.async_copy(tab_hbm.at[:, pl.ds(off, BW)], blk.at[par],
                             sems[par])

    def drain(b, par):
        @pl.when(jnp.logical_and(b < BPT, b0 + b < NBLK))
        def _():
            pltpu.make_async_copy(tab_hbm.at[:, pl.ds(0, BW)], blk.at[par],
                                  sems[par]).wait()

    def scatter():
        pltpu.async_copy(stg, out_hbm.at[eid.at[0]], sems[2]).wait()

    def sanitize(cnt):
        def body_s(i, c):
            pos = i * L + lanes
            cur = eid[0, pl.ds(i * L, L)]
            eid[0, pl.ds(i * L, L)] = jnp.where(pos >= cnt,
                                                nrows + pos, cur)
            return c
        lax.fori_loop(0, FLUSH // L, body_s, 0)

    def extract(par, p, cnt):
        v16 = sv[pl.ds(p, L)]
        e16 = ev[pl.ds(p, L)]
        c = v16[0] & jnp.int32(BW - 1)
        cv = jnp.full((L,), c, jnp.int32)
        for k in range(D // L):
            g = plsc.load_gather(blk.at[par], [k * L + lanes, cv])
            stg[cnt, pl.ds(k * L, L)] = g
        eid[0, pl.ds(cnt, L)] = e16
        cnt = cnt + 1

        @pl.when(cnt == FLUSH)
        def _():
            scatter()

        return jnp.where(cnt == FLUSH, 0, cnt)

    def one_block(gg, b, par, cnt):
        drain(b, par)
        fire(b + 2, par)
        sv16 = segv[pl.ds(2 * gg, L)]
        s0 = sv16[par]
        s1 = sv16[par + 1]

        def body_e(p, c):
            return extract(par, p, c)

        return lax.fori_loop(s0, s1, body_e, cnt)

    def body_seg(q, carry):
        e = (b0 + q * L + lanes) * jnp.int32(BW)
        lo = jnp.zeros((L,), jnp.int32)
        hi = jnp.full((L,), n_sorted + L, jnp.int32)

        def body_bs(it, lh):
            lo2, hi2 = lh
            mid = (lo2 + hi2) >> 1
            v = plsc.load_gather(sv, [mid])
            pred = v < e
            return (jnp.where(pred, mid + 1, lo2), jnp.where(pred, hi2, mid))

        lo, hi = lax.fori_loop(0, n_iters, body_bs, (lo, hi))
        segv[pl.ds(q * L, L)] = lo
        return carry

    lax.fori_loop(0, SEGW // L, body_seg, 0)

    fire(jnp.int32(0), 0)
    fire(jnp.int32(1), 1)

    def body_pair(gg, cnt):
        cnt = one_block(gg, 2 * gg, 0, cnt)
        cnt = one_block(gg, 2 * gg + 1, 1, cnt)
        return cnt

    cnt = lax.fori_loop(0, BPT // 2, body_pair, jnp.int32(0))
    sanitize(cnt)
    scatter()


def _sc_harvest(su, eu, si, ei, eu_hbm, ei_hbm,
                urows_hbm, irows_hbm,
                sv, ev, sgv, blk, stg, eid, sem0, sem1, sem2):
    sems = (sem0, sem1, sem2)

    pltpu.sync_copy(su, sv.at[pl.ds(0, B + L)])
    pltpu.sync_copy(eu, ev.at[pl.ds(0, B + L)])
    _harvest_pass(eu_hbm, B, sv, ev, sgv, blk, stg, eid, urows_hbm, sems, 15)

    pltpu.sync_copy(si, sv)
    pltpu.sync_copy(ei, ev)
    _harvest_pass(ei_hbm, 2 * B, sv, ev, sgv, blk, stg, eid, irows_hbm,
                  sems, 16)


_sc_harvest_call = functools.partial(
    pl.kernel,
    out_type=(jax.ShapeDtypeStruct((B + FLUSH, 2 * D), jnp.float32),
              jax.ShapeDtypeStruct((2 * B + FLUSH, 2 * D), jnp.float32)),
    mesh=plsc.VectorSubcoreMesh(core_axis_name="c", subcore_axis_name="s",
                                num_cores=NC, num_subcores=NS),
    scratch_types=[
        pltpu.VMEM((2 * B + L,), jnp.int32),
        pltpu.VMEM((2 * B + L,), jnp.int32),
        pltpu.VMEM((SEGW,), jnp.int32),
        pltpu.VMEM((2, D, BW), jnp.float32),
        pltpu.VMEM((FLUSH, 2 * D), jnp.float32),
        pltpu.VMEM((2, FLUSH), jnp.int32),
        pltpu.SemaphoreType.DMA,
        pltpu.SemaphoreType.DMA,
        pltpu.SemaphoreType.DMA,
    ],
    compiler_params=pltpu.CompilerParams(needs_layout_passes=False),
    name="bpr_sc_harvest",
)(_sc_harvest)


_TCB = 1024


def _tc_loss_body(u_ref, p_ref, n_ref, o_ref):
    i = pl.program_id(0)
    u = u_ref[:, :D]
    d = p_ref[:, :D] - n_ref[:, :D]
    s = jnp.sum(u * d, axis=-1)
    x = -s
    m = jnp.maximum(x, 0.0)
    sp = m + jnp.log(1.0 + jnp.exp(-jnp.abs(x)))
    part = jnp.sum(sp) * (1.0 / B)

    @pl.when(i == 0)
    def _():
        o_ref[0, 0] = 0.0

    o_ref[0, 0] += part


_tc_loss_call = pl.pallas_call(
    _tc_loss_body,
    out_shape=jax.ShapeDtypeStruct((1, 1), jnp.float32),
    grid=(B // _TCB,),
    in_specs=[
        pl.BlockSpec((_TCB, 2 * D), lambda i: (i, 0)),
        pl.BlockSpec((_TCB, 2 * D), lambda i: (i, 0)),
        pl.BlockSpec((_TCB, 2 * D), lambda i: (B // _TCB + i, 0)),
    ],
    out_specs=pl.BlockSpec(memory_space=pltpu.SMEM),
)


@jax.jit
def kernel(user, pos, neg, labels, embedding_user, embedding_item):
    del labels
    user = user.astype(jnp.int32)
    pos = pos.astype(jnp.int32)
    neg = neg.astype(jnp.int32)

    su, ou = lax.sort_key_val(user, jnp.arange(B, dtype=jnp.int32))
    itv = jnp.concatenate([pos, neg])
    si, oi = lax.sort_key_val(itv, jnp.arange(2 * B, dtype=jnp.int32))


    pad_s = jnp.full((L,), jnp.int32(SENT), jnp.int32)
    su = jnp.concatenate([su, pad_s, jnp.full((B,), jnp.int32(SENT), jnp.int32)])
    ou = jnp.concatenate([ou, pad_s, jnp.full((B,), jnp.int32(SENT), jnp.int32)])
    si = jnp.concatenate([si, pad_s])
    oi = jnp.concatenate([oi, pad_s])

    u_rows, i_rows = _sc_harvest_call(su[:B + L], ou[:B + L], si, oi,
                                      embedding_user.T, embedding_item.T)
    loss = _tc_loss_call(u_rows, i_rows, i_rows)
    return loss[0, 0]
